# Initial kernel scaffold; baseline (speedup 1.0000x reference)
import jax
import jax.numpy as jnp
from jax.experimental import pallas as pl


def kernel(h, edge_index, W1, b1, W2, b2):
    raise NotImplementedError("write your pallas kernel here")


# speedup vs baseline: 4.5737x; 4.5737x over previous
import functools

import jax
import jax.numpy as jnp
from jax import lax
from jax.experimental import pallas as pl
from jax.experimental.pallas import tpu as pltpu
from jax.experimental.pallas import tpu_sc as plsc

N = 10000
E = 320000
D = 128

NC = 2
NS = 16
NW = NC * NS

EPW = E // NW
CH = 80
NCH = EPW // CH
RPT = 624
REM = N - NS * RPT
REM_OFF = NS * RPT


def _sc_aggregate(h, src, dst):
  mesh = plsc.VectorSubcoreMesh(core_axis_name="c", subcore_axis_name="s")

  @functools.partial(
      pl.kernel,
      mesh=mesh,
      out_type=jax.ShapeDtypeStruct((NC, N, D), jnp.float32),
      scratch_types=[
          pltpu.VMEM((CH,), jnp.int32),
          pltpu.VMEM((CH,), jnp.int32),
          pltpu.VMEM((CH, D), jnp.float32),
          pltpu.VMEM_SHARED((N, D), jnp.float32),
          pltpu.SemaphoreType.DMA,
      ],
  )
  def agg_kernel(h_hbm, src_hbm, dst_hbm, out_hbm, src_v, dst_v, rows_v,
                 acc_sh, sem):
    cid = lax.axis_index("c")
    sid = lax.axis_index("s")
    wid = sid * NC + cid

    pltpu.sync_copy(h_hbm.at[pl.ds(sid * RPT, RPT)],
                    acc_sh.at[pl.ds(sid * RPT, RPT)])

    @pl.when(sid == 0)
    def _():
      pltpu.sync_copy(h_hbm.at[pl.ds(REM_OFF, REM)],
                      acc_sh.at[pl.ds(REM_OFF, REM)])

    plsc.subcore_barrier()

    ebase = wid * EPW

    def body(i, carry):
      off = ebase + i * CH
      pltpu.sync_copy(src_hbm.at[pl.ds(off, CH)], src_v)
      pltpu.sync_copy(dst_hbm.at[pl.ds(off, CH)], dst_v)
      pltpu.async_copy(h_hbm.at[src_v], rows_v, sem).wait()
      pltpu.sync_copy(rows_v, acc_sh.at[dst_v], add=True)
      return carry

    lax.fori_loop(0, NCH, body, 0)
    plsc.subcore_barrier()

    pltpu.sync_copy(acc_sh.at[pl.ds(sid * RPT, RPT)],
                    out_hbm.at[cid, pl.ds(sid * RPT, RPT)])

    @pl.when(sid == 0)
    def _():
      pltpu.sync_copy(acc_sh.at[pl.ds(REM_OFF, REM)],
                      out_hbm.at[cid, pl.ds(REM_OFF, REM)])

  return agg_kernel(h, src, dst)


def _tc_layer(x, pa, pb, W, b, final):
  BR = 1000

  def body(x_ref, a_ref, b_ref, w_ref, bias_ref, o_ref):
    rst = a_ref[...] + b_ref[...] - x_ref[...]
    y = jnp.dot(rst, w_ref[...], preferred_element_type=jnp.float32)
    y = jnp.maximum(y + bias_ref[...], 0.0)
    if final:
      m = jnp.max(y, axis=-1, keepdims=True)
      s = jnp.sum(jnp.exp(y - m), axis=-1, keepdims=True)
      y = y - (m + jnp.log(s))
    o_ref[...] = y

  row_spec = pl.BlockSpec((BR, D), lambda i: (i, 0))
  return pl.pallas_call(
      body,
      grid=(N // BR,),
      in_specs=[
          row_spec,
          row_spec,
          row_spec,
          pl.BlockSpec((D, D), lambda i: (0, 0)),
          pl.BlockSpec((1, D), lambda i: (0, 0)),
      ],
      out_specs=row_spec,
      out_shape=jax.ShapeDtypeStruct((N, D), jnp.float32),
  )(x, pa, pb, W, b)


def kernel(h, edge_index, W1, b1, W2, b2):
  src = edge_index[0]
  dst = edge_index[1]
  b1r = b1.reshape(1, D)
  b2r = b2.reshape(1, D)

  p = _sc_aggregate(h, src, dst)
  h1 = _tc_layer(h, p[0], p[1], W1, b1r, final=False)
  p2 = _sc_aggregate(h1, src, dst)
  return _tc_layer(h1, p2[0], p2[1], W2, b2r, final=True)


# speedup vs baseline: 9.6054x; 2.1002x over previous
import functools

import jax
import jax.numpy as jnp
from jax import lax
from jax.experimental import pallas as pl
from jax.experimental.pallas import tpu as pltpu
from jax.experimental.pallas import tpu_sc as plsc

N = 10000
E = 320000
D = 128

NC = 2
NS = 16
NW = NC * NS

EPW = E // NW
CH = 80
NCH = EPW // CH
RPT = 624
REM = N - NS * RPT
REM_OFF = NS * RPT


def _sc_aggregate(h, src, dst):
  mesh = plsc.VectorSubcoreMesh(core_axis_name="c", subcore_axis_name="s")

  @functools.partial(
      pl.kernel,
      mesh=mesh,
      out_type=jax.ShapeDtypeStruct((NC, N, D), jnp.float32),
      scratch_types=[
          pltpu.VMEM((EPW,), jnp.int32),
          pltpu.VMEM((CH,), jnp.int32),
          pltpu.VMEM((CH,), jnp.int32),
          pltpu.VMEM((CH, D), jnp.float32),
          pltpu.VMEM((CH, D), jnp.float32),
          pltpu.VMEM_SHARED((N, D), jnp.float32),
          pltpu.SemaphoreType.DMA,
          pltpu.SemaphoreType.DMA,
      ],
  )
  def agg_kernel(h_hbm, src_hbm, dst_hbm, out_hbm, srcall_v,
                 dsta_v, dstb_v, rows_a, rows_b, acc_sh,
                 sem_a, sem_b):
    cid = lax.axis_index("c")
    sid = lax.axis_index("s")
    wid = sid * NC + cid

    pltpu.sync_copy(h_hbm.at[pl.ds(sid * RPT, RPT)],
                    acc_sh.at[pl.ds(sid * RPT, RPT)])

    @pl.when(sid == 0)
    def _():
      pltpu.sync_copy(h_hbm.at[pl.ds(REM_OFF, REM)],
                      acc_sh.at[pl.ds(REM_OFF, REM)])

    ebase = wid * EPW
    pltpu.sync_copy(src_hbm.at[pl.ds(ebase, EPW)], srcall_v)
    plsc.subcore_barrier()

    def gather(c, rows, sem):
      pltpu.async_copy(h_hbm.at[srcall_v.at[pl.ds(c * CH, CH)]], rows, sem)

    def drain(rows, sem):
      pltpu.make_async_copy(h_hbm.at[pl.ds(0, CH)], rows, sem).wait()

    def scat(c, rows, dstv, sem):
      pltpu.sync_copy(dst_hbm.at[pl.ds(ebase + c * CH, CH)], dstv)
      drain(rows, sem)
      pltpu.sync_copy(rows, acc_sh.at[dstv], add=True)

    gather(0, rows_a, sem_a)

    def body(g, carry):
      c = 2 * g
      gather(c + 1, rows_b, sem_b)
      scat(c, rows_a, dsta_v, sem_a)
      gather(c + 2, rows_a, sem_a)
      scat(c + 1, rows_b, dstb_v, sem_b)
      return carry

    lax.fori_loop(0, (NCH - 1) // 2, body, 0)
    scat(NCH - 1, rows_a, dsta_v, sem_a)
    plsc.subcore_barrier()

    pltpu.sync_copy(acc_sh.at[pl.ds(sid * RPT, RPT)],
                    out_hbm.at[cid, pl.ds(sid * RPT, RPT)])

    @pl.when(sid == 0)
    def _():
      pltpu.sync_copy(acc_sh.at[pl.ds(REM_OFF, REM)],
                      out_hbm.at[cid, pl.ds(REM_OFF, REM)])

  return agg_kernel(h, src, dst)


def _tc_layer(x, pa, pb, W, b, final):
  BR = 1000

  def body(x_ref, a_ref, b_ref, w_ref, bias_ref, o_ref):
    rst = a_ref[...] + b_ref[...] - x_ref[...]
    y = jnp.dot(rst, w_ref[...], preferred_element_type=jnp.float32)
    y = jnp.maximum(y + bias_ref[...], 0.0)
    if final:
      m = jnp.max(y, axis=-1, keepdims=True)
      s = jnp.sum(jnp.exp(y - m), axis=-1, keepdims=True)
      y = y - (m + jnp.log(s))
    o_ref[...] = y

  row_spec = pl.BlockSpec((BR, D), lambda i: (i, 0))
  return pl.pallas_call(
      body,
      grid=(N // BR,),
      in_specs=[
          row_spec,
          row_spec,
          row_spec,
          pl.BlockSpec((D, D), lambda i: (0, 0)),
          pl.BlockSpec((1, D), lambda i: (0, 0)),
      ],
      out_specs=row_spec,
      out_shape=jax.ShapeDtypeStruct((N, D), jnp.float32),
  )(x, pa, pb, W, b)


def kernel(h, edge_index, W1, b1, W2, b2):
  src = edge_index[0]
  dst = edge_index[1]
  b1r = b1.reshape(1, D)
  b2r = b2.reshape(1, D)

  p = _sc_aggregate(h, src, dst)
  h1 = _tc_layer(h, p[0], p[1], W1, b1r, final=False)
  p2 = _sc_aggregate(h1, src, dst)
  return _tc_layer(h1, p2[0], p2[1], W2, b2r, final=True)
